# baseline (speedup 1.0000x reference)
import jax
import jax.numpy as jnp
from jax.experimental import pallas as pl
from jax.experimental.pallas import tpu as pltpu

N = 16384
IN_DIM = 2048
HIDDEN = 1024
E = 16
TOPK = 2
DT = 0.02

BM = 2048


def _router_kernel(x_ref, u_ref, ub_ref, g_ref, gb_ref,
                   w_ref, i_ref, p_ref, xb_ref):
    del xb_ref
    acc = None
    kc = 512
    for k in range(IN_DIM // kc):
        sl = pl.ds(k * kc, kc)
        part = jax.lax.dot_general(
            x_ref[:, sl].astype(jnp.bfloat16), u_ref[:, sl],
            dimension_numbers=(((1,), (1,)), ((), ())),
            preferred_element_type=jnp.float32,
        )
        acc = part if acc is None else acc + part
    h = DT * jnp.tanh(acc + ub_ref[...])
    lt = jax.lax.dot_general(
        g_ref[...], h,
        dimension_numbers=(((1,), (1,)), ((), ())),
        preferred_element_type=jnp.float32,
    ) + gb_ref[...]
    m = jnp.max(lt, axis=0, keepdims=True)
    e = jnp.exp(lt - m)
    s = jnp.sum(e, axis=0, keepdims=True)
    probs = e / s
    p_ref[...] = probs

    row = jax.lax.broadcasted_iota(jnp.int32, probs.shape, 0)
    p1 = jnp.max(probs, axis=0, keepdims=True)
    i1 = jnp.min(jnp.where(probs == p1, row, E), axis=0, keepdims=True)
    masked = jnp.where(row == i1, -1.0, probs)
    p2 = jnp.max(masked, axis=0, keepdims=True)
    i2 = jnp.min(jnp.where(masked == p2, row, E), axis=0, keepdims=True)
    denom = p1 + p2 + 1e-08
    w_ref[...] = jnp.concatenate([p1 / denom, p2 / denom], axis=0)
    i_ref[...] = jnp.concatenate([i1, i2], axis=0)


def kernel(x, W_w, W_b, U_w, U_b, V_w, V_b, g_w, g_b):
    del W_w, V_w, V_b
    bias = (W_b + U_b).reshape(1, HIDDEN)
    gbt = g_b.reshape(E, 1)
    U_w = U_w.astype(jnp.bfloat16)
    grid = (N // BM,)
    weights_t, indices_t, probs_t = pl.pallas_call(
        _router_kernel,
        grid=grid,
        in_specs=[
            pl.BlockSpec((BM, IN_DIM), lambda i: (i, 0)),
            pl.BlockSpec((HIDDEN, IN_DIM), lambda i: (0, 0)),
            pl.BlockSpec((1, HIDDEN), lambda i: (0, 0)),
            pl.BlockSpec((E, HIDDEN), lambda i: (0, 0)),
            pl.BlockSpec((E, 1), lambda i: (0, 0)),
        ],
        out_specs=[
            pl.BlockSpec((TOPK, BM), lambda i: (0, i)),
            pl.BlockSpec((TOPK, BM), lambda i: (0, i)),
            pl.BlockSpec((E, BM), lambda i: (0, i)),
        ],
        out_shape=[
            jax.ShapeDtypeStruct((TOPK, N), jnp.float32),
            jax.ShapeDtypeStruct((TOPK, N), jnp.int32),
            jax.ShapeDtypeStruct((E, N), jnp.float32),
        ],
        scratch_shapes=[pltpu.VMEM((BM, IN_DIM), jnp.bfloat16)],
    )(x, U_w, bias, g_w, gbt)
    return weights_t.T, indices_t.T, probs_t.T


# speedup vs baseline: 1.0296x; 1.0296x over previous
import jax
import jax.numpy as jnp
from jax.experimental import pallas as pl
from jax.experimental.pallas import tpu as pltpu

N = 16384
IN_DIM = 2048
HIDDEN = 1024
E = 16
TOPK = 2
DT = 0.02

BM = 1024


def _router_kernel(x_ref, u_ref, ub_ref, g_ref, gb_ref,
                   w_ref, i_ref, p_ref, xb_ref):
    xb_ref[...] = x_ref[...].astype(jnp.bfloat16)
    acc = jax.lax.dot_general(
        xb_ref[...], u_ref[...],
        dimension_numbers=(((1,), (1,)), ((), ())),
        preferred_element_type=jnp.float32,
    )
    h = DT * jnp.tanh(acc + ub_ref[...])
    lt = jax.lax.dot_general(
        g_ref[...], h,
        dimension_numbers=(((1,), (1,)), ((), ())),
        preferred_element_type=jnp.float32,
    ) + gb_ref[...]
    m = jnp.max(lt, axis=0, keepdims=True)
    e = jnp.exp(lt - m)
    s = jnp.sum(e, axis=0, keepdims=True)
    probs = e / s
    p_ref[...] = probs

    row = jax.lax.broadcasted_iota(jnp.int32, probs.shape, 0)
    p1 = jnp.max(probs, axis=0, keepdims=True)
    i1 = jnp.min(jnp.where(probs == p1, row, E), axis=0, keepdims=True)
    masked = jnp.where(row == i1, -1.0, probs)
    p2 = jnp.max(masked, axis=0, keepdims=True)
    i2 = jnp.min(jnp.where(masked == p2, row, E), axis=0, keepdims=True)
    denom = p1 + p2 + 1e-08
    w_ref[...] = jnp.concatenate([p1 / denom, p2 / denom], axis=0)
    i_ref[...] = jnp.concatenate([i1, i2], axis=0)


def kernel(x, W_w, W_b, U_w, U_b, V_w, V_b, g_w, g_b):
    del W_w, V_w, V_b
    bias = (W_b + U_b).reshape(1, HIDDEN)
    gbt = g_b.reshape(E, 1)
    U_w = U_w.astype(jnp.bfloat16)
    grid = (N // BM,)
    weights_t, indices_t, probs_t = pl.pallas_call(
        _router_kernel,
        grid=grid,
        in_specs=[
            pl.BlockSpec((BM, IN_DIM), lambda i: (i, 0)),
            pl.BlockSpec((HIDDEN, IN_DIM), lambda i: (0, 0)),
            pl.BlockSpec((1, HIDDEN), lambda i: (0, 0)),
            pl.BlockSpec((E, HIDDEN), lambda i: (0, 0)),
            pl.BlockSpec((E, 1), lambda i: (0, 0)),
        ],
        out_specs=[
            pl.BlockSpec((TOPK, BM), lambda i: (0, i)),
            pl.BlockSpec((TOPK, BM), lambda i: (0, i)),
            pl.BlockSpec((E, BM), lambda i: (0, i)),
        ],
        out_shape=[
            jax.ShapeDtypeStruct((TOPK, N), jnp.float32),
            jax.ShapeDtypeStruct((TOPK, N), jnp.int32),
            jax.ShapeDtypeStruct((E, N), jnp.float32),
        ],
        scratch_shapes=[pltpu.VMEM((BM, IN_DIM), jnp.bfloat16)],
    )(x, U_w, bias, g_w, gbt)
    return weights_t.T, indices_t.T, probs_t.T
